# baseline (speedup 1.0000x reference)
import functools

import jax
import jax.numpy as jnp
from jax.experimental import pallas as pl
from jax.experimental.pallas import tpu as pltpu
from jax.experimental.pallas import tpu_sc as plsc

_DELTA = 0.001
_K = 50
_KPAD = 64
_N = 100000
_C = 512
_V = 256
_BLK = 2048
_NBLK = (_N + _BLK - 1) // _BLK


def _tc_body(key_ref, keys_ref, idx_ref, wrow_ref, tot_ref, w_s, flat_s):
    b = pl.program_id(0)

    kblk = keys_ref[...]
    q = key_ref[...]
    diff = kblk - q
    d = jnp.sum(diff * diff, axis=1)
    d2 = d.reshape(1, _BLK)
    w2 = 1.0 / (d2 + _DELTA)
    lio = jax.lax.broadcasted_iota(jnp.int32, (1, _BLK), 1)
    valid = (lio + b * _BLK) < _N
    w2 = jnp.where(valid, w2, -1.0)
    w_s[pl.ds(b, 1), :] = w2
    flat_s[pl.ds(b, 1), :] = lio + b * _BLK

    @pl.when(b == 0)
    def _init():
        tot_ref[...] = jnp.zeros_like(tot_ref)

    tot_ref[...] += jnp.sum(jnp.maximum(w2, 0.0), axis=1, keepdims=True)

    @pl.when(b == _NBLK - 1)
    def _select():
        idx_ref[...] = jnp.zeros((_KPAD, 1), jnp.int32)
        wrow_ref[...] = jnp.zeros((_KPAD, _V), jnp.float32)
        invt11 = 1.0 / tot_ref[...]
        m0 = jnp.max(w_s[...], axis=(0, 1), keepdims=True)

        def body(i, m11):
            w = w_s[...]
            flat = flat_s[...]
            big = jnp.int32(_NBLK * _BLK)
            sel11 = jnp.min(jnp.where(w == m11, flat, big),
                            axis=(0, 1), keepdims=True)
            idx_ref[pl.ds(i, 1), :] = sel11
            wrow_ref[pl.ds(i, 1), :] = jnp.broadcast_to(
                m11 * invt11, (1, _V))
            w_new = jnp.where(flat == sel11, -1.0, w)
            w_s[...] = w_new
            return jnp.max(w_new, axis=(0, 1), keepdims=True)

        jax.lax.fori_loop(0, _K, body, m0)


def _tc_topk(key2, keys):
    return pl.pallas_call(
        _tc_body,
        grid=(_NBLK,),
        in_specs=[
            pl.BlockSpec((1, _C), lambda b: (0, 0)),
            pl.BlockSpec((_BLK, _C), lambda b: (b, 0)),
        ],
        out_specs=[
            pl.BlockSpec((_KPAD, 1), lambda b: (0, 0)),
            pl.BlockSpec((_KPAD, _V), lambda b: (0, 0)),
        ],
        out_shape=[
            jax.ShapeDtypeStruct((_KPAD, 1), jnp.int32),
            jax.ShapeDtypeStruct((_KPAD, _V), jnp.float32),
        ],
        scratch_shapes=[
            pltpu.VMEM((1, 1), jnp.float32),
            pltpu.VMEM((_NBLK, _BLK), jnp.float32),
            pltpu.VMEM((_NBLK, _BLK), jnp.int32),
        ],
        compiler_params=pltpu.CompilerParams(
            dimension_semantics=("arbitrary",),
        ),
    )(key2, keys)


def _sc_gather_sum(values, idx, wrow):
    mesh = plsc.VectorSubcoreMesh(core_axis_name="c", subcore_axis_name="s",
                                  num_cores=1)

    @functools.partial(
        pl.kernel,
        out_type=jax.ShapeDtypeStruct((1, _V), jnp.float32),
        mesh=mesh,
        scratch_types=[
            pltpu.VMEM((_KPAD,), jnp.int32),
            pltpu.VMEM((_KPAD, _V), jnp.float32),
            pltpu.VMEM((_KPAD, _V), jnp.float32),
            pltpu.VMEM((16,), jnp.float32),
            pltpu.SemaphoreType.DMA,
            pltpu.SemaphoreType.DMA,
        ],
    )
    def sc_kernel(values_hbm, idx_hbm, w_hbm, out_hbm,
                  idx_v, rows_v, w_v, acc_v, sem, sem2):
        cid = jax.lax.axis_index("c")
        sid = jax.lax.axis_index("s")

        @pl.when(cid == 0)
        def _():
            base = sid * 16
            wcopy = pltpu.async_copy(w_hbm, w_v, sem2)
            pltpu.sync_copy(idx_hbm, idx_v)
            pltpu.async_copy(values_hbm.at[idx_v], rows_v, sem).wait()
            wcopy.wait()
            acc_v[...] = jnp.zeros((16,), jnp.float32)

            @pl.loop(0, _KPAD)
            def _(j):
                acc_v[...] += (rows_v[j, pl.ds(base, 16)]
                               * w_v[j, pl.ds(base, 16)])

            pltpu.sync_copy(acc_v, out_hbm.at[0, pl.ds(base, 16)])

    return sc_kernel(values, idx, wrow)


def kernel(key, keys, values):
    idx, wrow = _tc_topk(key.reshape(1, _C), keys)
    return _sc_gather_sum(values, idx.reshape(_KPAD), wrow)


# speedup vs baseline: 1.1349x; 1.1349x over previous
import functools

import jax
import jax.numpy as jnp
from jax.experimental import pallas as pl
from jax.experimental.pallas import tpu as pltpu
from jax.experimental.pallas import tpu_sc as plsc

_DELTA = 0.001
_K = 50
_KPAD = 56
_N = 100000
_C = 512
_V = 256
_BLK = 4096
_NBLK = (_N + _BLK - 1) // _BLK


def _tc_body(key_ref, keys_ref, idx_ref, wrow_ref, tot_ref, w_s, flat_s):
    b = pl.program_id(0)

    kblk = keys_ref[...]
    q = key_ref[...]
    diff = kblk - q
    d = jnp.sum(diff * diff, axis=1)
    d2 = d.reshape(1, _BLK)
    w2 = 1.0 / (d2 + _DELTA)
    lio = jax.lax.broadcasted_iota(jnp.int32, (1, _BLK), 1)
    valid = (lio + b * _BLK) < _N
    w2 = jnp.where(valid, w2, -1.0)
    w_s[pl.ds(b, 1), :] = w2
    flat_s[pl.ds(b, 1), :] = lio + b * _BLK

    @pl.when(b == 0)
    def _init():
        tot_ref[...] = jnp.zeros_like(tot_ref)

    tot_ref[...] += jnp.sum(jnp.maximum(w2, 0.0), axis=1, keepdims=True)

    @pl.when(b == _NBLK - 1)
    def _select():
        idx_ref[...] = jnp.zeros((_KPAD, 1), jnp.int32)
        wrow_ref[...] = jnp.zeros((_KPAD, _V), jnp.float32)
        invt11 = 1.0 / tot_ref[...]
        m0 = jnp.max(w_s[...], axis=(0, 1), keepdims=True)

        def body(i, m11):
            w = w_s[...]
            flat = flat_s[...]
            big = jnp.int32(_NBLK * _BLK)
            sel11 = jnp.min(jnp.where(w == m11, flat, big),
                            axis=(0, 1), keepdims=True)
            idx_ref[pl.ds(i, 1), :] = sel11
            wrow_ref[pl.ds(i, 1), :] = jnp.broadcast_to(
                m11 * invt11, (1, _V))
            w_new = jnp.where(flat == sel11, -1.0, w)
            w_s[...] = w_new
            return jnp.max(w_new, axis=(0, 1), keepdims=True)

        jax.lax.fori_loop(0, _K, body, m0)


def _tc_topk(key2, keys):
    return pl.pallas_call(
        _tc_body,
        grid=(_NBLK,),
        in_specs=[
            pl.BlockSpec((1, _C), lambda b: (0, 0)),
            pl.BlockSpec((_BLK, _C), lambda b: (b, 0)),
        ],
        out_specs=[
            pl.BlockSpec((_KPAD, 1), lambda b: (0, 0)),
            pl.BlockSpec((_KPAD, _V), lambda b: (0, 0)),
        ],
        out_shape=[
            jax.ShapeDtypeStruct((_KPAD, 1), jnp.int32),
            jax.ShapeDtypeStruct((_KPAD, _V), jnp.float32),
        ],
        scratch_shapes=[
            pltpu.VMEM((1, 1), jnp.float32),
            pltpu.VMEM((_NBLK, _BLK), jnp.float32),
            pltpu.VMEM((_NBLK, _BLK), jnp.int32),
        ],
        compiler_params=pltpu.CompilerParams(
            dimension_semantics=("arbitrary",),
        ),
    )(key2, keys)


def _sc_gather_sum(values, idx, wrow):
    mesh = plsc.VectorSubcoreMesh(core_axis_name="c", subcore_axis_name="s",
                                  num_cores=1)

    @functools.partial(
        pl.kernel,
        out_type=jax.ShapeDtypeStruct((1, _V), jnp.float32),
        mesh=mesh,
        scratch_types=[
            pltpu.VMEM((_KPAD,), jnp.int32),
            pltpu.VMEM((_KPAD, _V), jnp.float32),
            pltpu.VMEM((_KPAD, _V), jnp.float32),
            pltpu.VMEM((16,), jnp.float32),
            pltpu.SemaphoreType.DMA,
            pltpu.SemaphoreType.DMA,
        ],
    )
    def sc_kernel(values_hbm, idx_hbm, w_hbm, out_hbm,
                  idx_v, rows_v, w_v, acc_v, sem, sem2):
        cid = jax.lax.axis_index("c")
        sid = jax.lax.axis_index("s")

        @pl.when(cid == 0)
        def _():
            base = sid * 16
            wcopy = pltpu.async_copy(w_hbm, w_v, sem2)
            pltpu.sync_copy(idx_hbm, idx_v)
            pltpu.async_copy(values_hbm.at[idx_v], rows_v, sem).wait()
            wcopy.wait()
            acc_v[...] = jnp.zeros((16,), jnp.float32)

            @pl.loop(0, _KPAD)
            def _(j):
                acc_v[...] += (rows_v[j, pl.ds(base, 16)]
                               * w_v[j, pl.ds(base, 16)])

            pltpu.sync_copy(acc_v, out_hbm.at[0, pl.ds(base, 16)])

    return sc_kernel(values, idx, wrow)


def kernel(key, keys, values):
    idx, wrow = _tc_topk(key.reshape(1, _C), keys)
    return _sc_gather_sum(values, idx.reshape(_KPAD), wrow)
